# Initial kernel scaffold; baseline (speedup 1.0000x reference)
import jax
import jax.numpy as jnp
from jax.experimental import pallas as pl


def kernel(memory, last_update, events_features, time_w, time_b, timestamps, src_nodes, dst_nodes, event_indices, idx):
    raise NotImplementedError("write your pallas kernel here")


# speedup vs baseline: 3.3478x; 3.3478x over previous
import dataclasses
import functools

import jax
import jax.numpy as jnp
from jax import lax
from jax.experimental import pallas as pl
from jax.experimental.pallas import tpu as pltpu
from jax.experimental.pallas import tpu_sc as plsc

N_NODES = 10000
N_EVENTS = 320000
D_FEAT = 128
TIME_DIM = 32
D_EDGE = 16
D_TAIL = TIME_DIM + D_EDGE
D_OUT = 2 * D_FEAT + D_TAIL

NUM_CORES = 2
NUM_SUBCORES = 16
NUM_LANES = 16
NUM_WORKERS = NUM_CORES * NUM_SUBCORES
EV_PER_WORKER = N_EVENTS // NUM_WORKERS
WIN = 400
N_WIN = EV_PER_WORKER // WIN

_MESH = plsc.VectorSubcoreMesh(core_axis_name="c", subcore_axis_name="s")

_NO_LAYOUT_CP = pltpu.CompilerParams()
if "needs_layout_passes" in pltpu.CompilerParams.__dataclass_fields__:
    _NO_LAYOUT_CP = dataclasses.replace(_NO_LAYOUT_CP, needs_layout_passes=False)


def _sc_stage_a(last_update, feat_pad, idx, evt):
    @functools.partial(
        pl.kernel,
        out_type=(
            jax.ShapeDtypeStruct((N_EVENTS,), jnp.float32),
            jax.ShapeDtypeStruct((N_EVENTS, D_FEAT), jnp.float32),
        ),
        mesh=_MESH,
        scratch_types=[
            pltpu.VMEM((N_NODES,), jnp.float32),
            pltpu.VMEM((EV_PER_WORKER,), jnp.int32),
            pltpu.VMEM((EV_PER_WORKER,), jnp.float32),
            pltpu.VMEM((WIN,), jnp.int32),
            pltpu.VMEM((WIN, D_FEAT), jnp.float32),
            pltpu.SemaphoreType.DMA,
        ],
        compiler_params=_NO_LAYOUT_CP,
    )
    def k(lu_hbm, feat_hbm, idx_hbm, evt_hbm, t_hbm, fo_hbm,
          lu_v, idx_v, t_v, ei_v, frows_v, sem):
        wid = lax.axis_index("s") * NUM_CORES + lax.axis_index("c")
        base = wid * EV_PER_WORKER
        pltpu.sync_copy(lu_hbm, lu_v)
        pltpu.sync_copy(idx_hbm.at[pl.ds(base, EV_PER_WORKER)], idx_v)

        @pl.loop(0, EV_PER_WORKER, step=NUM_LANES)
        def _(i):
            v = idx_v[pl.ds(i, NUM_LANES)]
            t_v[pl.ds(i, NUM_LANES)] = plsc.load_gather(lu_v, [v])

        pltpu.sync_copy(t_v, t_hbm.at[pl.ds(base, EV_PER_WORKER)])

        @pl.loop(0, N_WIN)
        def _(win):
            wbase = base + win * WIN
            pltpu.sync_copy(evt_hbm.at[pl.ds(wbase, WIN)], ei_v)
            pltpu.async_copy(feat_hbm.at[ei_v], frows_v, sem).wait()
            pltpu.sync_copy(frows_v, fo_hbm.at[pl.ds(wbase, WIN), :])

    return k(last_update, feat_pad, idx, evt)


_BT = 2000


def _tail_body(ts_ref, t_ref, w_ref, b_ref, f_ref, o_ref):
    dt = ts_ref[...] - t_ref[...]
    delta = jnp.cos(dt * w_ref[...] + b_ref[...])
    o_ref[...] = jnp.concatenate(
        [delta, f_ref[:, :D_EDGE]], axis=1)


def _tc_tail(ts2, t2, w2, b2, feats):
    return pl.pallas_call(
        _tail_body,
        grid=(N_EVENTS // _BT,),
        in_specs=[
            pl.BlockSpec((_BT, 1), lambda i: (i, 0)),
            pl.BlockSpec((_BT, 1), lambda i: (i, 0)),
            pl.BlockSpec((1, TIME_DIM), lambda i: (0, 0)),
            pl.BlockSpec((1, TIME_DIM), lambda i: (0, 0)),
            pl.BlockSpec((_BT, D_FEAT), lambda i: (i, 0)),
        ],
        out_specs=pl.BlockSpec((_BT, D_TAIL), lambda i: (i, 0)),
        out_shape=jax.ShapeDtypeStruct((N_EVENTS, D_TAIL), jnp.float32),
    )(ts2, t2, w2, b2, feats)


def _sc_assemble(memory, tail, src, dst):
    @functools.partial(
        pl.kernel,
        out_type=jax.ShapeDtypeStruct((N_EVENTS, D_OUT), jnp.float32),
        mesh=_MESH,
        scratch_types=[
            pltpu.VMEM((WIN,), jnp.int32),
            pltpu.VMEM((WIN, D_FEAT), jnp.float32),
            pltpu.VMEM((WIN, D_TAIL), jnp.float32),
            pltpu.SemaphoreType.DMA,
        ],
    )
    def k(mem_hbm, tail_hbm, src_hbm, dst_hbm, out_hbm, idx_v, rows_v, tl_v, sem):
        wid = lax.axis_index("s") * NUM_CORES + lax.axis_index("c")

        @pl.loop(0, N_WIN)
        def _(win):
            base = wid * EV_PER_WORKER + win * WIN
            pltpu.sync_copy(src_hbm.at[pl.ds(base, WIN)], idx_v)
            pltpu.async_copy(mem_hbm.at[idx_v], rows_v, sem).wait()
            pltpu.sync_copy(rows_v, out_hbm.at[pl.ds(base, WIN), pl.ds(0, D_FEAT)])
            pltpu.sync_copy(dst_hbm.at[pl.ds(base, WIN)], idx_v)
            pltpu.async_copy(mem_hbm.at[idx_v], rows_v, sem).wait()
            pltpu.sync_copy(rows_v, out_hbm.at[pl.ds(base, WIN), pl.ds(D_FEAT, D_FEAT)])
            pltpu.sync_copy(tail_hbm.at[pl.ds(base, WIN), :], tl_v)
            pltpu.sync_copy(tl_v, out_hbm.at[pl.ds(base, WIN), pl.ds(2 * D_FEAT, D_TAIL)])

    return k(memory, tail, src, dst)


def kernel(memory, last_update, events_features, time_w, time_b, timestamps,
           src_nodes, dst_nodes, event_indices, idx):
    feat_pad = jnp.pad(events_features, ((0, 0), (0, D_FEAT - D_EDGE)))
    t, feats = _sc_stage_a(
        last_update, feat_pad, idx.astype(jnp.int32),
        event_indices.astype(jnp.int32))
    ts2 = timestamps.reshape(N_EVENTS, 1)
    w2 = time_w.reshape(1, TIME_DIM)
    b2 = time_b.reshape(1, TIME_DIM)
    tail = _tc_tail(ts2, t.reshape(N_EVENTS, 1), w2, b2, feats)
    return _sc_assemble(
        memory, tail,
        src_nodes.astype(jnp.int32), dst_nodes.astype(jnp.int32))


# speedup vs baseline: 4.3757x; 1.3070x over previous
import dataclasses
import functools

import jax
import jax.numpy as jnp
from jax import lax
from jax.experimental import pallas as pl
from jax.experimental.pallas import tpu as pltpu
from jax.experimental.pallas import tpu_sc as plsc

N_NODES = 10000
N_EVENTS = 320000
D_FEAT = 128
TIME_DIM = 32
D_EDGE = 16

NUM_CORES = 2
NUM_SUBCORES = 16
NUM_LANES = 16
NUM_WORKERS = NUM_CORES * NUM_SUBCORES
EV_PER_WORKER = N_EVENTS // NUM_WORKERS
WIN = 400
N_WIN = EV_PER_WORKER // WIN

_MESH = plsc.VectorSubcoreMesh(core_axis_name="c", subcore_axis_name="s")


def _sc_cp(**kw):
    cp = pltpu.CompilerParams()
    fields = pltpu.CompilerParams.__dataclass_fields__
    return dataclasses.replace(cp, **{k: v for k, v in kw.items() if k in fields})


def _sc_stage_a(last_update, events_features, timestamps, idx, evt):
    @functools.partial(
        pl.kernel,
        out_type=(
            jax.ShapeDtypeStruct((N_EVENTS,), jnp.float32),
            jax.ShapeDtypeStruct((N_EVENTS, D_EDGE), jnp.float32),
        ),
        mesh=_MESH,
        scratch_types=[
            pltpu.VMEM((N_NODES,), jnp.float32),
            pltpu.VMEM((EV_PER_WORKER,), jnp.int32),
            pltpu.VMEM((EV_PER_WORKER,), jnp.float32),
            pltpu.VMEM((EV_PER_WORKER,), jnp.float32),
            pltpu.VMEM((WIN,), jnp.int32),
            pltpu.VMEM((WIN, D_EDGE), jnp.float32),
            pltpu.SemaphoreType.DMA,
        ],
        compiler_params=_sc_cp(needs_layout_passes=False,
                               use_tc_tiling_on_sc=False),
    )
    def k(lu_hbm, feat_hbm, ts_hbm, idx_hbm, evt_hbm, dt_hbm, ff_hbm,
          lu_v, idx_v, ts_v, dt_v, ei_v, frows_v, sem):
        wid = lax.axis_index("s") * NUM_CORES + lax.axis_index("c")
        base = wid * EV_PER_WORKER
        pltpu.sync_copy(lu_hbm, lu_v)
        pltpu.sync_copy(idx_hbm.at[pl.ds(base, EV_PER_WORKER)], idx_v)
        pltpu.sync_copy(ts_hbm.at[pl.ds(base, EV_PER_WORKER)], ts_v)

        @pl.loop(0, EV_PER_WORKER, step=NUM_LANES)
        def _(i):
            v = idx_v[pl.ds(i, NUM_LANES)]
            t16 = plsc.load_gather(lu_v, [v])
            dt_v[pl.ds(i, NUM_LANES)] = ts_v[pl.ds(i, NUM_LANES)] - t16

        pltpu.sync_copy(dt_v, dt_hbm.at[pl.ds(base, EV_PER_WORKER)])

        @pl.loop(0, N_WIN)
        def _(win):
            wbase = base + win * WIN
            pltpu.sync_copy(evt_hbm.at[pl.ds(wbase, WIN)], ei_v)
            pltpu.async_copy(feat_hbm.at[ei_v], frows_v, sem).wait()
            pltpu.sync_copy(frows_v, ff_hbm.at[pl.ds(wbase, WIN), :])

    return k(last_update, events_features, timestamps, idx, evt)


_BTC = 512


def _delta_body(dt_ref, w_ref, b_ref, o_ref):
    dtv = dt_ref[...].reshape(1, _BTC)
    o_ref[...] = jnp.cos(w_ref[...] * dtv + b_ref[...])


def _tc_delta_t(dt, w_col, b_col):
    return pl.pallas_call(
        _delta_body,
        grid=(N_EVENTS // _BTC,),
        in_specs=[
            pl.BlockSpec((_BTC,), lambda i: (i,)),
            pl.BlockSpec((TIME_DIM, 1), lambda i: (0, 0)),
            pl.BlockSpec((TIME_DIM, 1), lambda i: (0, 0)),
        ],
        out_specs=pl.BlockSpec((TIME_DIM, _BTC), lambda i: (0, i)),
        out_shape=jax.ShapeDtypeStruct((TIME_DIM, N_EVENTS), jnp.float32),
    )(dt, w_col, b_col)


def _sc_gather_mem(memory, src, dst):
    @functools.partial(
        pl.kernel,
        out_type=jax.ShapeDtypeStruct((N_EVENTS, 2 * D_FEAT), jnp.float32),
        mesh=_MESH,
        scratch_types=[
            pltpu.VMEM((WIN,), jnp.int32),
            pltpu.VMEM((WIN, D_FEAT), jnp.float32),
            pltpu.SemaphoreType.DMA,
        ],
    )
    def k(mem_hbm, src_hbm, dst_hbm, out_hbm, idx_v, rows_v, sem):
        wid = lax.axis_index("s") * NUM_CORES + lax.axis_index("c")

        @pl.loop(0, N_WIN)
        def _(win):
            base = wid * EV_PER_WORKER + win * WIN
            pltpu.sync_copy(src_hbm.at[pl.ds(base, WIN)], idx_v)
            pltpu.async_copy(mem_hbm.at[idx_v], rows_v, sem).wait()
            pltpu.sync_copy(rows_v, out_hbm.at[pl.ds(base, WIN), pl.ds(0, D_FEAT)])
            pltpu.sync_copy(dst_hbm.at[pl.ds(base, WIN)], idx_v)
            pltpu.async_copy(mem_hbm.at[idx_v], rows_v, sem).wait()
            pltpu.sync_copy(rows_v, out_hbm.at[pl.ds(base, WIN), pl.ds(D_FEAT, D_FEAT)])

    return k(memory, src, dst)


def kernel(memory, last_update, events_features, time_w, time_b, timestamps,
           src_nodes, dst_nodes, event_indices, idx):
    dt, ff = _sc_stage_a(
        last_update, events_features, timestamps,
        idx.astype(jnp.int32), event_indices.astype(jnp.int32))
    delta_t = _tc_delta_t(
        dt, time_w.reshape(TIME_DIM, 1), time_b.reshape(TIME_DIM, 1))
    out256 = _sc_gather_mem(
        memory, src_nodes.astype(jnp.int32), dst_nodes.astype(jnp.int32))
    return jnp.concatenate([out256, delta_t.T, ff], axis=1)


# speedup vs baseline: 4.6849x; 1.0707x over previous
import dataclasses
import functools

import jax
import jax.numpy as jnp
from jax import lax
from jax.experimental import pallas as pl
from jax.experimental.pallas import tpu as pltpu
from jax.experimental.pallas import tpu_sc as plsc

N_NODES = 10000
N_EVENTS = 320000
D_FEAT = 128
TIME_DIM = 32
D_EDGE = 16

NUM_CORES = 2
NUM_SUBCORES = 16
NUM_LANES = 16
NUM_WORKERS = NUM_CORES * NUM_SUBCORES
EV_PER_WORKER = N_EVENTS // NUM_WORKERS
WIN = 400
N_WIN = EV_PER_WORKER // WIN

_MESH = plsc.VectorSubcoreMesh(core_axis_name="c", subcore_axis_name="s")


def _sc_cp(**kw):
    cp = pltpu.CompilerParams()
    fields = pltpu.CompilerParams.__dataclass_fields__
    return dataclasses.replace(cp, **{k: v for k, v in kw.items() if k in fields})


def _sc_stage_a(last_update, events_features, timestamps, idx, evt):
    @functools.partial(
        pl.kernel,
        out_type=(
            jax.ShapeDtypeStruct((N_EVENTS,), jnp.float32),
            jax.ShapeDtypeStruct((N_EVENTS, D_EDGE), jnp.float32),
        ),
        mesh=_MESH,
        scratch_types=[
            pltpu.VMEM((N_NODES,), jnp.float32),
            pltpu.VMEM((EV_PER_WORKER,), jnp.int32),
            pltpu.VMEM((EV_PER_WORKER,), jnp.float32),
            pltpu.VMEM((EV_PER_WORKER,), jnp.float32),
            pltpu.VMEM((WIN,), jnp.int32),
            pltpu.VMEM((WIN, D_EDGE), jnp.float32),
            pltpu.SemaphoreType.DMA,
        ],
        compiler_params=_sc_cp(needs_layout_passes=False,
                               use_tc_tiling_on_sc=False),
    )
    def k(lu_hbm, feat_hbm, ts_hbm, idx_hbm, evt_hbm, dt_hbm, ff_hbm,
          lu_v, idx_v, ts_v, dt_v, ei_v, frows_v, sem):
        wid = lax.axis_index("s") * NUM_CORES + lax.axis_index("c")
        base = wid * EV_PER_WORKER
        pltpu.sync_copy(lu_hbm, lu_v)
        pltpu.sync_copy(idx_hbm.at[pl.ds(base, EV_PER_WORKER)], idx_v)
        pltpu.sync_copy(ts_hbm.at[pl.ds(base, EV_PER_WORKER)], ts_v)

        @pl.loop(0, EV_PER_WORKER, step=NUM_LANES)
        def _(i):
            v = idx_v[pl.ds(i, NUM_LANES)]
            t16 = plsc.load_gather(lu_v, [v])
            dt_v[pl.ds(i, NUM_LANES)] = ts_v[pl.ds(i, NUM_LANES)] - t16

        pltpu.sync_copy(dt_v, dt_hbm.at[pl.ds(base, EV_PER_WORKER)])

        @pl.loop(0, N_WIN)
        def _(win):
            wbase = base + win * WIN
            pltpu.sync_copy(evt_hbm.at[pl.ds(wbase, WIN)], ei_v)
            pltpu.async_copy(feat_hbm.at[ei_v], frows_v, sem).wait()
            pltpu.sync_copy(frows_v, ff_hbm.at[pl.ds(wbase, WIN), :])

    return k(last_update, events_features, timestamps, idx, evt)


_BTC = 12800


def _delta_body(dt_ref, w_ref, b_ref, o_ref):
    i = pl.program_id(0)
    dtv = dt_ref[pl.ds(i * _BTC, _BTC)].reshape(1, _BTC)
    o_ref[...] = jnp.cos(w_ref[...] * dtv + b_ref[...])


def _tc_delta_t(dt, w_col, b_col):
    return pl.pallas_call(
        _delta_body,
        grid=(N_EVENTS // _BTC,),
        in_specs=[
            pl.BlockSpec((N_EVENTS,), lambda i: (0,)),
            pl.BlockSpec((TIME_DIM, 1), lambda i: (0, 0)),
            pl.BlockSpec((TIME_DIM, 1), lambda i: (0, 0)),
        ],
        out_specs=pl.BlockSpec((TIME_DIM, _BTC), lambda i: (0, i)),
        out_shape=jax.ShapeDtypeStruct((TIME_DIM, N_EVENTS), jnp.float32),
    )(dt, w_col, b_col)


def _sc_gather_mem(memory, src, dst):
    @functools.partial(
        pl.kernel,
        out_type=jax.ShapeDtypeStruct((N_EVENTS, 2 * D_FEAT), jnp.float32),
        mesh=_MESH,
        scratch_types=[
            pltpu.VMEM((WIN,), jnp.int32),
            pltpu.VMEM((WIN, D_FEAT), jnp.float32),
            pltpu.SemaphoreType.DMA,
        ],
    )
    def k(mem_hbm, src_hbm, dst_hbm, out_hbm, idx_v, rows_v, sem):
        wid = lax.axis_index("s") * NUM_CORES + lax.axis_index("c")

        @pl.loop(0, N_WIN)
        def _(win):
            base = wid * EV_PER_WORKER + win * WIN
            pltpu.sync_copy(src_hbm.at[pl.ds(base, WIN)], idx_v)
            pltpu.async_copy(mem_hbm.at[idx_v], rows_v, sem).wait()
            pltpu.sync_copy(rows_v, out_hbm.at[pl.ds(base, WIN), pl.ds(0, D_FEAT)])
            pltpu.sync_copy(dst_hbm.at[pl.ds(base, WIN)], idx_v)
            pltpu.async_copy(mem_hbm.at[idx_v], rows_v, sem).wait()
            pltpu.sync_copy(rows_v, out_hbm.at[pl.ds(base, WIN), pl.ds(D_FEAT, D_FEAT)])

    return k(memory, src, dst)


def kernel(memory, last_update, events_features, time_w, time_b, timestamps,
           src_nodes, dst_nodes, event_indices, idx):
    dt, ff = _sc_stage_a(
        last_update, events_features, timestamps,
        idx.astype(jnp.int32), event_indices.astype(jnp.int32))
    delta_t = _tc_delta_t(
        dt, time_w.reshape(TIME_DIM, 1), time_b.reshape(TIME_DIM, 1))
    out256 = _sc_gather_mem(
        memory, src_nodes.astype(jnp.int32), dst_nodes.astype(jnp.int32))
    out_t = jnp.concatenate([out256.T, delta_t, ff.T], axis=0)
    return out_t.T
